# baseline (speedup 1.0000x reference)
import functools

import jax
import jax.numpy as jnp
from jax import lax
from jax.experimental import pallas as pl
from jax.experimental.pallas import tpu as pltpu
from jax.experimental.pallas import tpu_sc as plsc

_N = 10000
_E = 320000
_DF = 128
_DE = 16
_H = 96
_G = 64
_BN_EPS = 1e-5

_TILES = 32
_CH = 80
_EPT = 10240
_EPAD = _TILES * _EPT
_NIT = _EPT // _CH
_NITA = 80
_NITB = 2 * _NIT - _NITA
_NP = 10112
_RPT = _NP // 16


def _edge_linear(ea, wbs):
    BLK = 4096
    grid = _EPAD // BLK

    def body(ea_ref, w_ref, b_ref, o_ref):
        o_ref[...] = jnp.dot(ea_ref[...], w_ref[...],
                             preferred_element_type=jnp.float32) + b_ref[...]

    def full(shape):
        return pl.BlockSpec(shape, lambda i: (0, 0))

    outs = []
    for (w, b) in wbs:
        ew = w.shape[1]
        outs.append(pl.pallas_call(
            body,
            grid=(grid,),
            in_specs=[pl.BlockSpec((BLK, _DE), lambda i: (i, 0)),
                      full((_DE, ew)), full((1, ew))],
            out_specs=pl.BlockSpec((BLK, ew), lambda i: (i, 0)),
            out_shape=jax.ShapeDtypeStruct((_EPAD, ew), jnp.float32),
        )(ea, w, b))
    return outs


def _make_msgpass(ew):
    mesh = plsc.VectorSubcoreMesh(core_axis_name="c", subcore_axis_name="s")
    HCH = _CH // 2

    @functools.partial(
        pl.kernel,
        out_type=jax.ShapeDtypeStruct((2, _NP, _DF), jnp.float32),
        mesh=mesh,
        scratch_types=[
            [pltpu.VMEM((2, _CH), jnp.int32) for _ in range(4)],
            [pltpu.VMEM((_CH, _DF), jnp.float32) for _ in range(2)],
            [pltpu.VMEM((_CH, ew), jnp.float32) for _ in range(2)],
            pltpu.VMEM_SHARED((_NP, _DF), jnp.float32),
            [pltpu.SemaphoreType.DMA for _ in range(4)],
            [pltpu.SemaphoreType.DMA for _ in range(2)],
            [pltpu.SemaphoreType.DMA for _ in range(2)],
            [pltpu.SemaphoreType.DMA for _ in range(2)],
            [pltpu.SemaphoreType.DMA for _ in range(2)],
        ],
    )
    def msgpass(idx_hbm, e_hbm, x_hbm, z_hbm, out_hbm,
                idx_v, m_v, e_v, acc_sh, sem_i, sem_e, sem_g, sem_g2, sem_s):
        c = lax.axis_index("c")
        s = lax.axis_index("s")
        row0 = s * _RPT
        nit = jnp.where(c == 0, _NITA, _NITB)
        cid0 = jnp.where(c == 0, s * _NITA, 16 * _NITA + s * _NITB)
        pltpu.sync_copy(z_hbm.at[pl.ds(row0, _RPT)], acc_sh.at[pl.ds(row0, _RPT)])
        plsc.subcore_barrier()

        pltpu.sync_copy(idx_hbm.at[cid0], idx_v[0])
        pltpu.sync_copy(idx_hbm.at[cid0 + 1], idx_v[1])
        pltpu.async_copy(e_hbm.at[pl.ds(cid0 * _CH, _CH)], e_v[0], sem_e[0])
        pltpu.async_copy(x_hbm.at[idx_v[0].at[0, pl.ds(0, HCH)]],
                         m_v[0].at[pl.ds(0, HCH)], sem_g[0])
        pltpu.async_copy(x_hbm.at[idx_v[0].at[0, pl.ds(HCH, HCH)]],
                         m_v[0].at[pl.ds(HCH, HCH)], sem_g2[0])

        def group(g, carry):
            for u in range(4):
                i = g * 4 + u
                bm = u % 2
                bn = (u + 1) % 2
                bi = u % 4
                bi1 = (u + 1) % 4
                bi2 = (u + 2) % 4
                bi_1 = (u + 3) % 4
                pltpu.make_async_copy(e_hbm.at[pl.ds(0, _CH)], e_v[bm],
                                      sem_e[bm]).wait()
                pltpu.make_async_copy(x_hbm.at[idx_v[bi].at[0, pl.ds(0, HCH)]],
                                      m_v[bm].at[pl.ds(0, HCH)],
                                      sem_g[bm]).wait()
                pltpu.make_async_copy(x_hbm.at[idx_v[bi].at[0, pl.ds(0, HCH)]],
                                      m_v[bm].at[pl.ds(HCH, HCH)],
                                      sem_g2[bm]).wait()

                @pl.when(i >= 1)
                def _():
                    pltpu.make_async_copy(m_v[bn], acc_sh.at[idx_v[bi_1].at[1]],
                                          sem_s[bn]).wait()

                @pl.when(i + 1 < nit)
                def _():
                    @pl.when(i + 1 >= 2)
                    def _():
                        pltpu.make_async_copy(idx_hbm.at[cid0], idx_v[bi1],
                                              sem_i[bi1]).wait()
                    pltpu.async_copy(x_hbm.at[idx_v[bi1].at[0, pl.ds(0, HCH)]],
                                     m_v[bn].at[pl.ds(0, HCH)], sem_g[bn])
                    pltpu.async_copy(x_hbm.at[idx_v[bi1].at[0, pl.ds(HCH, HCH)]],
                                     m_v[bn].at[pl.ds(HCH, HCH)], sem_g2[bn])
                    pltpu.async_copy(
                        e_hbm.at[pl.ds((cid0 + i + 1) * _CH, _CH)],
                        e_v[bn], sem_e[bn])

                    @pl.when(i + 2 < nit)
                    def _():
                        pltpu.async_copy(idx_hbm.at[cid0 + i + 2],
                                         idx_v[bi2], sem_i[bi2])

                @plsc.parallel_loop(0, _CH, 1, unroll=2)
                def _(j):
                    for k in range(ew // 16):
                        sl = pl.ds(k * 16, 16)
                        m_v[bm][j, sl] = jnp.maximum(
                            m_v[bm][j, sl] + e_v[bm][j, sl], 0.0)

                pltpu.async_copy(m_v[bm], acc_sh.at[idx_v[bi].at[1]],
                                 sem_s[bm], add=True)
            return carry

        lax.fori_loop(0, nit // 4, group, 0)
        pltpu.make_async_copy(m_v[(_NITA - 1) % 2],
                              acc_sh.at[idx_v[(_NITA - 1) % 4].at[1]],
                              sem_s[(_NITA - 1) % 2]).wait()
        plsc.subcore_barrier()
        pltpu.sync_copy(acc_sh.at[pl.ds(row0, _RPT)],
                        out_hbm.at[c, pl.ds(row0, _RPT)])

    return msgpass


def _node_mlp(x, aggr2, w1, b1, w2, b2, oc):
    BLK = 2000
    grid = _N // BLK

    def body(x_ref, a0_ref, a1_ref, w1_ref, b1_ref, w2_ref, b2_ref, o_ref):
        hsum = x_ref[...] + a0_ref[0] + a1_ref[0]
        t = jnp.maximum(
            jnp.dot(hsum, w1_ref[...], preferred_element_type=jnp.float32) + b1_ref[...], 0.0)
        o_ref[...] = jnp.maximum(
            jnp.dot(t, w2_ref[...], preferred_element_type=jnp.float32) + b2_ref[...], 0.0)

    def full(shape):
        return pl.BlockSpec(shape, lambda i: tuple(0 for _ in shape))

    return pl.pallas_call(
        body,
        grid=(grid,),
        in_specs=[pl.BlockSpec((BLK, _DF), lambda i: (i, 0)),
                  pl.BlockSpec((1, BLK, _DF), lambda i: (0, i, 0)),
                  pl.BlockSpec((1, BLK, _DF), lambda i: (1, i, 0)),
                  full((_DF, _H)), full((1, _H)),
                  full((_H, oc)), full((1, oc))],
        out_specs=pl.BlockSpec((BLK, oc), lambda i: (i, 0)),
        out_shape=jax.ShapeDtypeStruct((_N, oc), jnp.float32),
    )(x, aggr2, aggr2, w1, b1, w2, b2)


def _pool_classify(h, batch2d, cw1a, cw1b, cb1, cw2, cb2):
    BLK = 1000
    grid = _N // BLK
    NEG = -3.0e38

    def body(h_ref, b_ref, cw1a_ref, cw1b_ref, cb1_ref, cw2_ref, cb2_ref,
             out_ref, sum_ref, cnt_ref, mx_ref):
        i = pl.program_id(0)

        @pl.when(i == 0)
        def _():
            sum_ref[...] = jnp.zeros_like(sum_ref)
            cnt_ref[...] = jnp.zeros_like(cnt_ref)
            mx_ref[...] = jnp.full_like(mx_ref, NEG)

        hb = h_ref[...]
        bv = b_ref[...]
        onehot = (bv == lax.broadcasted_iota(jnp.int32, (BLK, _G), 1)
                  ).astype(jnp.float32)
        sum_ref[...] += lax.dot_general(
            onehot, hb, (((0,), (0,)), ((), ())), preferred_element_type=jnp.float32)
        cnt_ref[...] += lax.dot_general(
            onehot, jnp.ones_like(hb), (((0,), (0,)), ((), ())),
            preferred_element_type=jnp.float32)

        g0 = jnp.min(bv)
        g1 = jnp.max(bv)

        def mbody(g, carry):
            vals = jnp.where(bv == g, hb, NEG)
            m = jnp.max(vals, axis=0, keepdims=True)
            mx_ref[pl.ds(g, 1), :] = jnp.maximum(mx_ref[pl.ds(g, 1), :], m)
            return carry

        lax.fori_loop(g0, g1 + 1, mbody, 0)

        @pl.when(i == grid - 1)
        def _():
            cnt = cnt_ref[...]
            mean = sum_ref[...] / jnp.maximum(cnt, 1.0)
            mx = jnp.where(cnt > 0, mx_ref[...], 0.0)
            z = jnp.maximum(
                jnp.dot(mean, cw1a_ref[...], preferred_element_type=jnp.float32)
                + jnp.dot(mx, cw1b_ref[...], preferred_element_type=jnp.float32)
                + cb1_ref[...], 0.0)
            out_ref[...] = (jnp.dot(z, cw2_ref[...], preferred_element_type=jnp.float32)
                            + cb2_ref[...])

    def full(shape):
        return pl.BlockSpec(shape, lambda i: tuple(0 for _ in shape))

    return pl.pallas_call(
        body,
        grid=(grid,),
        in_specs=[pl.BlockSpec((BLK, _H), lambda i: (i, 0)),
                  pl.BlockSpec((BLK, 1), lambda i: (i, 0)),
                  full((_H, _H)), full((_H, _H)), full((1, _H)),
                  full((_H, 2)), full((1, 2))],
        out_specs=pl.BlockSpec((_G, 2), lambda i: (0, 0)),
        out_shape=jax.ShapeDtypeStruct((_G, 2), jnp.float32),
        scratch_shapes=[pltpu.VMEM((_G, _H), jnp.float32),
                        pltpu.VMEM((_G, _H), jnp.float32),
                        pltpu.VMEM((_G, _H), jnp.float32)],
    )(h, batch2d, cw1a, cw1b, cb1, cw2, cb2)


def kernel(x, edge_index, edge_attr, batch,
           c1_le_w, c1_le_b, c1_w1, c1_b1, c1_bn_g, c1_bn_b, c1_w2, c1_b2,
           c2_le_w, c2_le_b, c2_w1, c2_b1, c2_bn_g, c2_bn_b, c2_w2, c2_b2,
           c3_le_w, c3_le_b, c3_w1, c3_b1, c3_bn_g, c3_bn_b, c3_w2, c3_b2,
           cl_w1, cl_b1, cl_w2, cl_b2):
    padE = _EPAD - _E
    src_p = jnp.concatenate([edge_index[0], jnp.zeros((padE,), jnp.int32)])
    dst_p = jnp.concatenate([edge_index[1], jnp.full((padE,), _N, jnp.int32)])
    ea_p = jnp.concatenate([edge_attr, jnp.zeros((padE, _DE), jnp.float32)])
    idx2 = jnp.stack([src_p.reshape(-1, _CH), dst_p.reshape(-1, _CH)], axis=1)

    bnscale = jnp.float32(1.0 / (1.0 + _BN_EPS) ** 0.5)

    def padcols(w, n):
        return jnp.pad(w, ((0, 0), (0, n - w.shape[-1])))

    def padrows(w, n):
        return jnp.pad(w, ((0, n - w.shape[0]), (0, 0)))

    layers = []
    for li, (le_w, le_b, w1, b1, bn_g, bn_b, w2, b2) in enumerate((
            (c1_le_w, c1_le_b, c1_w1, c1_b1, c1_bn_g, c1_bn_b, c1_w2, c1_b2),
            (c2_le_w, c2_le_b, c2_w1, c2_b1, c2_bn_g, c2_bn_b, c2_w2, c2_b2),
            (c3_le_w, c3_le_b, c3_w1, c3_b1, c3_bn_g, c3_bn_b, c3_w2, c3_b2))):
        g = bn_g * bnscale
        w1f = w1 * g[None, :]
        b1f = (b1 * g + bn_b).reshape(1, -1)
        oc = _DF if li < 2 else _H
        ew = _DF if li == 0 else _H
        layers.append((padcols(le_w, ew), padcols(le_b.reshape(1, -1), ew),
                       padrows(w1f, _DF), b1f,
                       padcols(w2, oc), padcols(b2.reshape(1, -1), oc), oc, ew))

    e1, e2, e3 = _edge_linear(ea_p, [(layers[0][0], layers[0][1]),
                                     (layers[1][0], layers[1][1]),
                                     (layers[2][0], layers[2][1])])

    zeros = jnp.zeros((_NP, _DF), jnp.float32)
    h = x
    for (e_l, lay) in ((e1, layers[0]), (e2, layers[1]), (e3, layers[2])):
        aggr2 = _make_msgpass(lay[7])(idx2, e_l, h, zeros)
        h = _node_mlp(h, aggr2, lay[2], lay[3], lay[4], lay[5], lay[6])

    return _pool_classify(h, batch.reshape(_N, 1),
                          cl_w1[:_H], cl_w1[_H:], cl_b1.reshape(1, -1),
                          cl_w2, cl_b2.reshape(1, -1))


# speedup vs baseline: 1.1129x; 1.1129x over previous
import functools

import jax
import jax.numpy as jnp
from jax import lax
from jax.experimental import pallas as pl
from jax.experimental.pallas import tpu as pltpu
from jax.experimental.pallas import tpu_sc as plsc

_N = 10000
_E = 320000
_DF = 128
_DE = 16
_H = 96
_G = 64
_BN_EPS = 1e-5

_TILES = 32
_CH = 80
_EPT = 10240
_EPAD = _TILES * _EPT
_NIT = _EPT // _CH
_NITA = 176
_NITB = 2 * _NIT - _NITA
_NP = 10112
_RPT = _NP // 16


def _edge_linear(ea, wbs):
    BLK = 4096
    grid = _EPAD // BLK

    def body(ea_ref, w_ref, b_ref, o_ref):
        o_ref[...] = jnp.dot(ea_ref[...], w_ref[...],
                             preferred_element_type=jnp.float32) + b_ref[...]

    def full(shape):
        return pl.BlockSpec(shape, lambda i: (0, 0))

    outs = []
    for (w, b) in wbs:
        ew = w.shape[1]
        outs.append(pl.pallas_call(
            body,
            grid=(grid,),
            in_specs=[pl.BlockSpec((BLK, _DE), lambda i: (i, 0)),
                      full((_DE, ew)), full((1, ew))],
            out_specs=pl.BlockSpec((BLK, ew), lambda i: (i, 0)),
            out_shape=jax.ShapeDtypeStruct((_EPAD, ew), jnp.float32),
        )(ea, w, b))
    return outs


def _make_msgpass(ew):
    mesh = plsc.VectorSubcoreMesh(core_axis_name="c", subcore_axis_name="s")
    HCH = _CH // 2

    @functools.partial(
        pl.kernel,
        out_type=jax.ShapeDtypeStruct((2, _NP, _DF), jnp.float32),
        mesh=mesh,
        scratch_types=[
            [pltpu.VMEM((2, _CH), jnp.int32) for _ in range(4)],
            [pltpu.VMEM((_CH, _DF), jnp.float32) for _ in range(2)],
            [pltpu.VMEM((_CH, ew), jnp.float32) for _ in range(2)],
            pltpu.VMEM_SHARED((_NP, _DF), jnp.float32),
            [pltpu.SemaphoreType.DMA for _ in range(4)],
            [pltpu.SemaphoreType.DMA for _ in range(2)],
            [pltpu.SemaphoreType.DMA for _ in range(2)],
            [pltpu.SemaphoreType.DMA for _ in range(2)],
            [pltpu.SemaphoreType.DMA for _ in range(2)],
        ],
    )
    def msgpass(idx_hbm, e_hbm, x_hbm, z_hbm, out_hbm,
                idx_v, m_v, e_v, acc_sh, sem_i, sem_e, sem_g, sem_g2, sem_s):
        c = lax.axis_index("c")
        s = lax.axis_index("s")
        row0 = s * _RPT
        nit = jnp.where(c == 0, _NITA, _NITB)
        cid0 = jnp.where(c == 0, s * _NITA, 16 * _NITA + s * _NITB)
        pltpu.sync_copy(z_hbm.at[pl.ds(row0, _RPT)], acc_sh.at[pl.ds(row0, _RPT)])
        plsc.subcore_barrier()

        pltpu.sync_copy(idx_hbm.at[cid0], idx_v[0])
        pltpu.sync_copy(idx_hbm.at[cid0 + 1], idx_v[1])
        pltpu.async_copy(e_hbm.at[pl.ds(cid0 * _CH, _CH)], e_v[0], sem_e[0])
        pltpu.async_copy(x_hbm.at[idx_v[0].at[0, pl.ds(0, HCH)]],
                         m_v[0].at[pl.ds(0, HCH)], sem_g[0])
        pltpu.async_copy(x_hbm.at[idx_v[0].at[0, pl.ds(HCH, HCH)]],
                         m_v[0].at[pl.ds(HCH, HCH)], sem_g2[0])

        def group(g, carry):
            for u in range(4):
                i = g * 4 + u
                bm = u % 2
                bn = (u + 1) % 2
                bi = u % 4
                bi1 = (u + 1) % 4
                bi2 = (u + 2) % 4
                bi_1 = (u + 3) % 4
                pltpu.make_async_copy(e_hbm.at[pl.ds(0, _CH)], e_v[bm],
                                      sem_e[bm]).wait()
                pltpu.make_async_copy(x_hbm.at[idx_v[bi].at[0, pl.ds(0, HCH)]],
                                      m_v[bm].at[pl.ds(0, HCH)],
                                      sem_g[bm]).wait()
                pltpu.make_async_copy(x_hbm.at[idx_v[bi].at[0, pl.ds(0, HCH)]],
                                      m_v[bm].at[pl.ds(HCH, HCH)],
                                      sem_g2[bm]).wait()

                @pl.when(i >= 1)
                def _():
                    pltpu.make_async_copy(m_v[bn], acc_sh.at[idx_v[bi_1].at[1]],
                                          sem_s[bn]).wait()

                @pl.when(i + 1 < nit)
                def _():
                    @pl.when(i + 1 >= 2)
                    def _():
                        pltpu.make_async_copy(idx_hbm.at[cid0], idx_v[bi1],
                                              sem_i[bi1]).wait()
                    pltpu.async_copy(x_hbm.at[idx_v[bi1].at[0, pl.ds(0, HCH)]],
                                     m_v[bn].at[pl.ds(0, HCH)], sem_g[bn])
                    pltpu.async_copy(x_hbm.at[idx_v[bi1].at[0, pl.ds(HCH, HCH)]],
                                     m_v[bn].at[pl.ds(HCH, HCH)], sem_g2[bn])
                    pltpu.async_copy(
                        e_hbm.at[pl.ds((cid0 + i + 1) * _CH, _CH)],
                        e_v[bn], sem_e[bn])

                    @pl.when(i + 2 < nit)
                    def _():
                        pltpu.async_copy(idx_hbm.at[cid0 + i + 2],
                                         idx_v[bi2], sem_i[bi2])

                @plsc.parallel_loop(0, _CH, 1, unroll=2)
                def _(j):
                    for k in range(ew // 16):
                        sl = pl.ds(k * 16, 16)
                        m_v[bm][j, sl] = jnp.maximum(
                            m_v[bm][j, sl] + e_v[bm][j, sl], 0.0)

                pltpu.async_copy(m_v[bm], acc_sh.at[idx_v[bi].at[1]],
                                 sem_s[bm], add=True)
            return carry

        lax.fori_loop(0, nit // 4, group, 0)
        pltpu.make_async_copy(m_v[(_NITA - 1) % 2],
                              acc_sh.at[idx_v[(_NITA - 1) % 4].at[1]],
                              sem_s[(_NITA - 1) % 2]).wait()
        plsc.subcore_barrier()
        pltpu.sync_copy(acc_sh.at[pl.ds(row0, _RPT)],
                        out_hbm.at[c, pl.ds(row0, _RPT)])

    return msgpass


def _node_mlp(x, aggr2, w1, b1, w2, b2, oc):
    BLK = 2000
    grid = _N // BLK

    def body(x_ref, a0_ref, a1_ref, w1_ref, b1_ref, w2_ref, b2_ref, o_ref):
        hsum = x_ref[...] + a0_ref[0] + a1_ref[0]
        t = jnp.maximum(
            jnp.dot(hsum, w1_ref[...], preferred_element_type=jnp.float32) + b1_ref[...], 0.0)
        o_ref[...] = jnp.maximum(
            jnp.dot(t, w2_ref[...], preferred_element_type=jnp.float32) + b2_ref[...], 0.0)

    def full(shape):
        return pl.BlockSpec(shape, lambda i: tuple(0 for _ in shape))

    return pl.pallas_call(
        body,
        grid=(grid,),
        in_specs=[pl.BlockSpec((BLK, _DF), lambda i: (i, 0)),
                  pl.BlockSpec((1, BLK, _DF), lambda i: (0, i, 0)),
                  pl.BlockSpec((1, BLK, _DF), lambda i: (1, i, 0)),
                  full((_DF, _H)), full((1, _H)),
                  full((_H, oc)), full((1, oc))],
        out_specs=pl.BlockSpec((BLK, oc), lambda i: (i, 0)),
        out_shape=jax.ShapeDtypeStruct((_N, oc), jnp.float32),
    )(x, aggr2, aggr2, w1, b1, w2, b2)


def _pool_classify(h, batch2d, cw1a, cw1b, cb1, cw2, cb2):
    BLK = 1000
    grid = _N // BLK
    NEG = -3.0e38

    def body(h_ref, b_ref, cw1a_ref, cw1b_ref, cb1_ref, cw2_ref, cb2_ref,
             out_ref, sum_ref, cnt_ref, mx_ref):
        i = pl.program_id(0)

        @pl.when(i == 0)
        def _():
            sum_ref[...] = jnp.zeros_like(sum_ref)
            cnt_ref[...] = jnp.zeros_like(cnt_ref)
            mx_ref[...] = jnp.full_like(mx_ref, NEG)

        hb = h_ref[...]
        bv = b_ref[...]
        onehot = (bv == lax.broadcasted_iota(jnp.int32, (BLK, _G), 1)
                  ).astype(jnp.float32)
        sum_ref[...] += lax.dot_general(
            onehot, hb, (((0,), (0,)), ((), ())), preferred_element_type=jnp.float32)
        cnt_ref[...] += lax.dot_general(
            onehot, jnp.ones_like(hb), (((0,), (0,)), ((), ())),
            preferred_element_type=jnp.float32)

        g0 = jnp.min(bv)
        g1 = jnp.max(bv)

        def mbody(g, carry):
            vals = jnp.where(bv == g, hb, NEG)
            m = jnp.max(vals, axis=0, keepdims=True)
            mx_ref[pl.ds(g, 1), :] = jnp.maximum(mx_ref[pl.ds(g, 1), :], m)
            return carry

        lax.fori_loop(g0, g1 + 1, mbody, 0)

        @pl.when(i == grid - 1)
        def _():
            cnt = cnt_ref[...]
            mean = sum_ref[...] / jnp.maximum(cnt, 1.0)
            mx = jnp.where(cnt > 0, mx_ref[...], 0.0)
            z = jnp.maximum(
                jnp.dot(mean, cw1a_ref[...], preferred_element_type=jnp.float32)
                + jnp.dot(mx, cw1b_ref[...], preferred_element_type=jnp.float32)
                + cb1_ref[...], 0.0)
            out_ref[...] = (jnp.dot(z, cw2_ref[...], preferred_element_type=jnp.float32)
                            + cb2_ref[...])

    def full(shape):
        return pl.BlockSpec(shape, lambda i: tuple(0 for _ in shape))

    return pl.pallas_call(
        body,
        grid=(grid,),
        in_specs=[pl.BlockSpec((BLK, _H), lambda i: (i, 0)),
                  pl.BlockSpec((BLK, 1), lambda i: (i, 0)),
                  full((_H, _H)), full((_H, _H)), full((1, _H)),
                  full((_H, 2)), full((1, 2))],
        out_specs=pl.BlockSpec((_G, 2), lambda i: (0, 0)),
        out_shape=jax.ShapeDtypeStruct((_G, 2), jnp.float32),
        scratch_shapes=[pltpu.VMEM((_G, _H), jnp.float32),
                        pltpu.VMEM((_G, _H), jnp.float32),
                        pltpu.VMEM((_G, _H), jnp.float32)],
    )(h, batch2d, cw1a, cw1b, cb1, cw2, cb2)


def kernel(x, edge_index, edge_attr, batch,
           c1_le_w, c1_le_b, c1_w1, c1_b1, c1_bn_g, c1_bn_b, c1_w2, c1_b2,
           c2_le_w, c2_le_b, c2_w1, c2_b1, c2_bn_g, c2_bn_b, c2_w2, c2_b2,
           c3_le_w, c3_le_b, c3_w1, c3_b1, c3_bn_g, c3_bn_b, c3_w2, c3_b2,
           cl_w1, cl_b1, cl_w2, cl_b2):
    padE = _EPAD - _E
    src_p = jnp.concatenate([edge_index[0], jnp.zeros((padE,), jnp.int32)])
    dst_p = jnp.concatenate([edge_index[1], jnp.full((padE,), _N, jnp.int32)])
    ea_p = jnp.concatenate([edge_attr, jnp.zeros((padE, _DE), jnp.float32)])
    idx2 = jnp.stack([src_p.reshape(-1, _CH), dst_p.reshape(-1, _CH)], axis=1)

    bnscale = jnp.float32(1.0 / (1.0 + _BN_EPS) ** 0.5)

    def padcols(w, n):
        return jnp.pad(w, ((0, 0), (0, n - w.shape[-1])))

    def padrows(w, n):
        return jnp.pad(w, ((0, n - w.shape[0]), (0, 0)))

    layers = []
    for li, (le_w, le_b, w1, b1, bn_g, bn_b, w2, b2) in enumerate((
            (c1_le_w, c1_le_b, c1_w1, c1_b1, c1_bn_g, c1_bn_b, c1_w2, c1_b2),
            (c2_le_w, c2_le_b, c2_w1, c2_b1, c2_bn_g, c2_bn_b, c2_w2, c2_b2),
            (c3_le_w, c3_le_b, c3_w1, c3_b1, c3_bn_g, c3_bn_b, c3_w2, c3_b2))):
        g = bn_g * bnscale
        w1f = w1 * g[None, :]
        b1f = (b1 * g + bn_b).reshape(1, -1)
        oc = _DF if li < 2 else _H
        ew = _DF if li == 0 else _H
        layers.append((padcols(le_w, ew), padcols(le_b.reshape(1, -1), ew),
                       padrows(w1f, _DF), b1f,
                       padcols(w2, oc), padcols(b2.reshape(1, -1), oc), oc, ew))

    e1, e2, e3 = _edge_linear(ea_p, [(layers[0][0], layers[0][1]),
                                     (layers[1][0], layers[1][1]),
                                     (layers[2][0], layers[2][1])])

    zeros = jnp.zeros((_NP, _DF), jnp.float32)
    h = x
    for (e_l, lay) in ((e1, layers[0]), (e2, layers[1]), (e3, layers[2])):
        aggr2 = _make_msgpass(lay[7])(idx2, e_l, h, zeros)
        h = _node_mlp(h, aggr2, lay[2], lay[3], lay[4], lay[5], lay[6])

    return _pool_classify(h, batch.reshape(_N, 1),
                          cl_w1[:_H], cl_w1[_H:], cl_b1.reshape(1, -1),
                          cl_w2, cl_b2.reshape(1, -1))


# speedup vs baseline: 1.1144x; 1.0014x over previous
import functools

import jax
import jax.numpy as jnp
from jax import lax
from jax.experimental import pallas as pl
from jax.experimental.pallas import tpu as pltpu
from jax.experimental.pallas import tpu_sc as plsc

_N = 10000
_E = 320000
_DF = 128
_DE = 16
_H = 96
_G = 64
_BN_EPS = 1e-5

_TILES = 32
_CH = 80
_EPT = 10240
_EPAD = _TILES * _EPT
_NIT = _EPT // _CH
_NITA = 176
_NITB = 2 * _NIT - _NITA
_NP = 10112
_RPT = _NP // 16


def _edge_linear(ea, wbs):
    BLK = 4096
    grid = _EPAD // BLK

    def body(ea_ref, w_ref, b_ref, o_ref):
        o_ref[...] = jnp.dot(ea_ref[...], w_ref[...],
                             preferred_element_type=jnp.float32) + b_ref[...]

    def full(shape):
        return pl.BlockSpec(shape, lambda i: (0, 0))

    outs = []
    for (w, b) in wbs:
        ew = w.shape[1]
        outs.append(pl.pallas_call(
            body,
            grid=(grid,),
            in_specs=[pl.BlockSpec((BLK, _DE), lambda i: (i, 0)),
                      full((_DE, ew)), full((1, ew))],
            out_specs=pl.BlockSpec((BLK, ew), lambda i: (i, 0)),
            out_shape=jax.ShapeDtypeStruct((_EPAD, ew), jnp.float32),
        )(ea, w, b))
    return outs


def _make_msgpass(ew):
    mesh = plsc.VectorSubcoreMesh(core_axis_name="c", subcore_axis_name="s")
    HCH = _CH // 2

    @functools.partial(
        pl.kernel,
        out_type=jax.ShapeDtypeStruct((2, _NP, _DF), jnp.float32),
        mesh=mesh,
        compiler_params=pltpu.CompilerParams(use_tc_tiling_on_sc=True),
        scratch_types=[
            [pltpu.VMEM((2, _CH), jnp.int32) for _ in range(4)],
            [pltpu.VMEM((_CH, _DF), jnp.float32) for _ in range(2)],
            [pltpu.VMEM((_CH, ew), jnp.float32) for _ in range(2)],
            pltpu.VMEM_SHARED((_NP, _DF), jnp.float32),
            [pltpu.SemaphoreType.DMA for _ in range(4)],
            [pltpu.SemaphoreType.DMA for _ in range(2)],
            [pltpu.SemaphoreType.DMA for _ in range(2)],
            [pltpu.SemaphoreType.DMA for _ in range(2)],
            [pltpu.SemaphoreType.DMA for _ in range(2)],
        ],
    )
    def msgpass(idx_hbm, e_hbm, x_hbm, z_hbm, out_hbm,
                idx_v, m_v, e_v, acc_sh, sem_i, sem_e, sem_g, sem_g2, sem_s):
        c = lax.axis_index("c")
        s = lax.axis_index("s")
        row0 = s * _RPT
        nit = jnp.where(c == 0, _NITA, _NITB)
        cid0 = jnp.where(c == 0, s * _NITA, 16 * _NITA + s * _NITB)
        pltpu.sync_copy(z_hbm.at[pl.ds(row0, _RPT)], acc_sh.at[pl.ds(row0, _RPT)])
        plsc.subcore_barrier()

        pltpu.sync_copy(idx_hbm.at[cid0], idx_v[0])
        pltpu.sync_copy(idx_hbm.at[cid0 + 1], idx_v[1])
        pltpu.async_copy(e_hbm.at[pl.ds(cid0 * _CH, _CH)], e_v[0], sem_e[0])
        pltpu.async_copy(x_hbm.at[idx_v[0].at[0, pl.ds(0, HCH)]],
                         m_v[0].at[pl.ds(0, HCH)], sem_g[0])
        pltpu.async_copy(x_hbm.at[idx_v[0].at[0, pl.ds(HCH, HCH)]],
                         m_v[0].at[pl.ds(HCH, HCH)], sem_g2[0])

        def group(g, carry):
            for u in range(4):
                i = g * 4 + u
                bm = u % 2
                bn = (u + 1) % 2
                bi = u % 4
                bi1 = (u + 1) % 4
                bi2 = (u + 2) % 4
                bi_1 = (u + 3) % 4
                pltpu.make_async_copy(e_hbm.at[pl.ds(0, _CH)], e_v[bm],
                                      sem_e[bm]).wait()
                pltpu.make_async_copy(x_hbm.at[idx_v[bi].at[0, pl.ds(0, HCH)]],
                                      m_v[bm].at[pl.ds(0, HCH)],
                                      sem_g[bm]).wait()
                pltpu.make_async_copy(x_hbm.at[idx_v[bi].at[0, pl.ds(0, HCH)]],
                                      m_v[bm].at[pl.ds(HCH, HCH)],
                                      sem_g2[bm]).wait()

                @pl.when(i >= 1)
                def _():
                    pltpu.make_async_copy(m_v[bn], acc_sh.at[idx_v[bi_1].at[1]],
                                          sem_s[bn]).wait()

                @pl.when(i + 1 < nit)
                def _():
                    @pl.when(i + 1 >= 2)
                    def _():
                        pltpu.make_async_copy(idx_hbm.at[cid0], idx_v[bi1],
                                              sem_i[bi1]).wait()
                    pltpu.async_copy(x_hbm.at[idx_v[bi1].at[0, pl.ds(0, HCH)]],
                                     m_v[bn].at[pl.ds(0, HCH)], sem_g[bn])
                    pltpu.async_copy(x_hbm.at[idx_v[bi1].at[0, pl.ds(HCH, HCH)]],
                                     m_v[bn].at[pl.ds(HCH, HCH)], sem_g2[bn])
                    pltpu.async_copy(
                        e_hbm.at[pl.ds((cid0 + i + 1) * _CH, _CH)],
                        e_v[bn], sem_e[bn])

                    @pl.when(i + 2 < nit)
                    def _():
                        pltpu.async_copy(idx_hbm.at[cid0 + i + 2],
                                         idx_v[bi2], sem_i[bi2])

                @plsc.parallel_loop(0, _CH, 1, unroll=2)
                def _(j):
                    for k in range(ew // 16):
                        sl = pl.ds(k * 16, 16)
                        m_v[bm][j, sl] = jnp.maximum(
                            m_v[bm][j, sl] + e_v[bm][j, sl], 0.0)

                pltpu.async_copy(m_v[bm], acc_sh.at[idx_v[bi].at[1]],
                                 sem_s[bm], add=True)
            return carry

        lax.fori_loop(0, nit // 4, group, 0)
        pltpu.make_async_copy(m_v[(_NITA - 1) % 2],
                              acc_sh.at[idx_v[(_NITA - 1) % 4].at[1]],
                              sem_s[(_NITA - 1) % 2]).wait()
        plsc.subcore_barrier()
        pltpu.sync_copy(acc_sh.at[pl.ds(row0, _RPT)],
                        out_hbm.at[c, pl.ds(row0, _RPT)])

    return msgpass


def _node_mlp(x, aggr2, w1, b1, w2, b2, oc):
    BLK = 2000
    grid = _N // BLK

    def body(x_ref, a0_ref, a1_ref, w1_ref, b1_ref, w2_ref, b2_ref, o_ref):
        hsum = x_ref[...] + a0_ref[0] + a1_ref[0]
        t = jnp.maximum(
            jnp.dot(hsum, w1_ref[...], preferred_element_type=jnp.float32) + b1_ref[...], 0.0)
        o_ref[...] = jnp.maximum(
            jnp.dot(t, w2_ref[...], preferred_element_type=jnp.float32) + b2_ref[...], 0.0)

    def full(shape):
        return pl.BlockSpec(shape, lambda i: tuple(0 for _ in shape))

    return pl.pallas_call(
        body,
        grid=(grid,),
        in_specs=[pl.BlockSpec((BLK, _DF), lambda i: (i, 0)),
                  pl.BlockSpec((1, BLK, _DF), lambda i: (0, i, 0)),
                  pl.BlockSpec((1, BLK, _DF), lambda i: (1, i, 0)),
                  full((_DF, _H)), full((1, _H)),
                  full((_H, oc)), full((1, oc))],
        out_specs=pl.BlockSpec((BLK, oc), lambda i: (i, 0)),
        out_shape=jax.ShapeDtypeStruct((_N, oc), jnp.float32),
    )(x, aggr2, aggr2, w1, b1, w2, b2)


def _pool_classify(h, batch2d, cw1a, cw1b, cb1, cw2, cb2):
    BLK = 1000
    grid = _N // BLK
    NEG = -3.0e38

    def body(h_ref, b_ref, cw1a_ref, cw1b_ref, cb1_ref, cw2_ref, cb2_ref,
             out_ref, sum_ref, cnt_ref, mx_ref):
        i = pl.program_id(0)

        @pl.when(i == 0)
        def _():
            sum_ref[...] = jnp.zeros_like(sum_ref)
            cnt_ref[...] = jnp.zeros_like(cnt_ref)
            mx_ref[...] = jnp.full_like(mx_ref, NEG)

        hb = h_ref[...]
        bv = b_ref[...]
        onehot = (bv == lax.broadcasted_iota(jnp.int32, (BLK, _G), 1)
                  ).astype(jnp.float32)
        sum_ref[...] += lax.dot_general(
            onehot, hb, (((0,), (0,)), ((), ())), preferred_element_type=jnp.float32)
        cnt_ref[...] += lax.dot_general(
            onehot, jnp.ones_like(hb), (((0,), (0,)), ((), ())),
            preferred_element_type=jnp.float32)

        g0 = jnp.min(bv)
        g1 = jnp.max(bv)

        def mbody(g, carry):
            vals = jnp.where(bv == g, hb, NEG)
            m = jnp.max(vals, axis=0, keepdims=True)
            mx_ref[pl.ds(g, 1), :] = jnp.maximum(mx_ref[pl.ds(g, 1), :], m)
            return carry

        lax.fori_loop(g0, g1 + 1, mbody, 0)

        @pl.when(i == grid - 1)
        def _():
            cnt = cnt_ref[...]
            mean = sum_ref[...] / jnp.maximum(cnt, 1.0)
            mx = jnp.where(cnt > 0, mx_ref[...], 0.0)
            z = jnp.maximum(
                jnp.dot(mean, cw1a_ref[...], preferred_element_type=jnp.float32)
                + jnp.dot(mx, cw1b_ref[...], preferred_element_type=jnp.float32)
                + cb1_ref[...], 0.0)
            out_ref[...] = (jnp.dot(z, cw2_ref[...], preferred_element_type=jnp.float32)
                            + cb2_ref[...])

    def full(shape):
        return pl.BlockSpec(shape, lambda i: tuple(0 for _ in shape))

    return pl.pallas_call(
        body,
        grid=(grid,),
        in_specs=[pl.BlockSpec((BLK, _H), lambda i: (i, 0)),
                  pl.BlockSpec((BLK, 1), lambda i: (i, 0)),
                  full((_H, _H)), full((_H, _H)), full((1, _H)),
                  full((_H, 2)), full((1, 2))],
        out_specs=pl.BlockSpec((_G, 2), lambda i: (0, 0)),
        out_shape=jax.ShapeDtypeStruct((_G, 2), jnp.float32),
        scratch_shapes=[pltpu.VMEM((_G, _H), jnp.float32),
                        pltpu.VMEM((_G, _H), jnp.float32),
                        pltpu.VMEM((_G, _H), jnp.float32)],
    )(h, batch2d, cw1a, cw1b, cb1, cw2, cb2)


def kernel(x, edge_index, edge_attr, batch,
           c1_le_w, c1_le_b, c1_w1, c1_b1, c1_bn_g, c1_bn_b, c1_w2, c1_b2,
           c2_le_w, c2_le_b, c2_w1, c2_b1, c2_bn_g, c2_bn_b, c2_w2, c2_b2,
           c3_le_w, c3_le_b, c3_w1, c3_b1, c3_bn_g, c3_bn_b, c3_w2, c3_b2,
           cl_w1, cl_b1, cl_w2, cl_b2):
    padE = _EPAD - _E
    src_p = jnp.concatenate([edge_index[0], jnp.zeros((padE,), jnp.int32)])
    dst_p = jnp.concatenate([edge_index[1], jnp.full((padE,), _N, jnp.int32)])
    ea_p = jnp.concatenate([edge_attr, jnp.zeros((padE, _DE), jnp.float32)])
    idx2 = jnp.stack([src_p.reshape(-1, _CH), dst_p.reshape(-1, _CH)], axis=1)

    bnscale = jnp.float32(1.0 / (1.0 + _BN_EPS) ** 0.5)

    def padcols(w, n):
        return jnp.pad(w, ((0, 0), (0, n - w.shape[-1])))

    def padrows(w, n):
        return jnp.pad(w, ((0, n - w.shape[0]), (0, 0)))

    layers = []
    for li, (le_w, le_b, w1, b1, bn_g, bn_b, w2, b2) in enumerate((
            (c1_le_w, c1_le_b, c1_w1, c1_b1, c1_bn_g, c1_bn_b, c1_w2, c1_b2),
            (c2_le_w, c2_le_b, c2_w1, c2_b1, c2_bn_g, c2_bn_b, c2_w2, c2_b2),
            (c3_le_w, c3_le_b, c3_w1, c3_b1, c3_bn_g, c3_bn_b, c3_w2, c3_b2))):
        g = bn_g * bnscale
        w1f = w1 * g[None, :]
        b1f = (b1 * g + bn_b).reshape(1, -1)
        oc = _DF if li < 2 else _H
        ew = _DF if li == 0 else _H
        layers.append((padcols(le_w, ew), padcols(le_b.reshape(1, -1), ew),
                       padrows(w1f, _DF), b1f,
                       padcols(w2, oc), padcols(b2.reshape(1, -1), oc), oc, ew))

    e1, e2, e3 = _edge_linear(ea_p, [(layers[0][0], layers[0][1]),
                                     (layers[1][0], layers[1][1]),
                                     (layers[2][0], layers[2][1])])

    zeros = jnp.zeros((_NP, _DF), jnp.float32)
    h = x
    for (e_l, lay) in ((e1, layers[0]), (e2, layers[1]), (e3, layers[2])):
        aggr2 = _make_msgpass(lay[7])(idx2, e_l, h, zeros)
        h = _node_mlp(h, aggr2, lay[2], lay[3], lay[4], lay[5], lay[6])

    return _pool_classify(h, batch.reshape(_N, 1),
                          cl_w1[:_H], cl_w1[_H:], cl_b1.reshape(1, -1),
                          cl_w2, cl_b2.reshape(1, -1))


# speedup vs baseline: 1.1353x; 1.0187x over previous
import functools

import jax
import jax.numpy as jnp
from jax import lax
from jax.experimental import pallas as pl
from jax.experimental.pallas import tpu as pltpu
from jax.experimental.pallas import tpu_sc as plsc

_N = 10000
_E = 320000
_DF = 128
_DE = 16
_H = 96
_G = 64
_BN_EPS = 1e-5

_TILES = 32
_CH = 80
_EPT = 10240
_EPAD = _TILES * _EPT
_NIT = _EPT // _CH
_NITA = 184
_NITB = 2 * _NIT - _NITA
_NP = 10112
_RPT = _NP // 16


def _edge_linear(ea, wbs):
    BLK = 4096
    grid = _EPAD // BLK

    def body(ea_ref, w_ref, b_ref, o_ref):
        o_ref[...] = jnp.dot(ea_ref[...], w_ref[...],
                             preferred_element_type=jnp.float32) + b_ref[...]

    def full(shape):
        return pl.BlockSpec(shape, lambda i: (0, 0))

    outs = []
    for (w, b) in wbs:
        ew = w.shape[1]
        outs.append(pl.pallas_call(
            body,
            grid=(grid,),
            in_specs=[pl.BlockSpec((BLK, _DE), lambda i: (i, 0)),
                      full((_DE, ew)), full((1, ew))],
            out_specs=pl.BlockSpec((BLK, ew), lambda i: (i, 0)),
            out_shape=jax.ShapeDtypeStruct((_EPAD, ew), jnp.float32),
        )(ea, w, b))
    return outs


def _make_msgpass(ew):
    mesh = plsc.VectorSubcoreMesh(core_axis_name="c", subcore_axis_name="s")
    HCH = _CH // 2

    @functools.partial(
        pl.kernel,
        out_type=jax.ShapeDtypeStruct((2, _NP, _DF), jnp.float32),
        mesh=mesh,
        compiler_params=pltpu.CompilerParams(use_tc_tiling_on_sc=True),
        scratch_types=[
            [pltpu.VMEM((2, _CH), jnp.int32) for _ in range(4)],
            [pltpu.VMEM((_CH, _DF), jnp.float32) for _ in range(2)],
            [pltpu.VMEM((_CH, ew), jnp.float32) for _ in range(2)],
            pltpu.VMEM_SHARED((_NP, _DF), jnp.float32),
            [pltpu.SemaphoreType.DMA for _ in range(4)],
            [pltpu.SemaphoreType.DMA for _ in range(2)],
            [pltpu.SemaphoreType.DMA for _ in range(2)],
            [pltpu.SemaphoreType.DMA for _ in range(2)],
            [pltpu.SemaphoreType.DMA for _ in range(2)],
        ],
    )
    def msgpass(idx_hbm, e_hbm, x_hbm, z_hbm, out_hbm,
                idx_v, m_v, e_v, acc_sh, sem_i, sem_e, sem_g, sem_g2, sem_s):
        c = lax.axis_index("c")
        s = lax.axis_index("s")
        row0 = s * _RPT
        nit = jnp.where(c == 0, _NITA, _NITB)
        cid0 = jnp.where(c == 0, s * _NITA, 16 * _NITA + s * _NITB)
        pltpu.sync_copy(z_hbm.at[pl.ds(row0, _RPT)], acc_sh.at[pl.ds(row0, _RPT)])
        plsc.subcore_barrier()

        pltpu.sync_copy(idx_hbm.at[cid0], idx_v[0])
        pltpu.sync_copy(idx_hbm.at[cid0 + 1], idx_v[1])
        pltpu.async_copy(e_hbm.at[pl.ds(cid0 * _CH, _CH)], e_v[0], sem_e[0])
        pltpu.async_copy(x_hbm.at[idx_v[0].at[0, pl.ds(0, HCH)]],
                         m_v[0].at[pl.ds(0, HCH)], sem_g[0])
        pltpu.async_copy(x_hbm.at[idx_v[0].at[0, pl.ds(HCH, HCH)]],
                         m_v[0].at[pl.ds(HCH, HCH)], sem_g2[0])

        def group(g, carry):
            for u in range(4):
                i = g * 4 + u
                bm = u % 2
                bn = (u + 1) % 2
                bi = u % 4
                bi1 = (u + 1) % 4
                bi2 = (u + 2) % 4
                bi_1 = (u + 3) % 4
                pltpu.make_async_copy(e_hbm.at[pl.ds(0, _CH)], e_v[bm],
                                      sem_e[bm]).wait()
                pltpu.make_async_copy(x_hbm.at[idx_v[bi].at[0, pl.ds(0, HCH)]],
                                      m_v[bm].at[pl.ds(0, HCH)],
                                      sem_g[bm]).wait()
                pltpu.make_async_copy(x_hbm.at[idx_v[bi].at[0, pl.ds(0, HCH)]],
                                      m_v[bm].at[pl.ds(HCH, HCH)],
                                      sem_g2[bm]).wait()

                @pl.when(i >= 1)
                def _():
                    pltpu.make_async_copy(m_v[bn], acc_sh.at[idx_v[bi_1].at[1]],
                                          sem_s[bn]).wait()

                @pl.when(i + 1 < nit)
                def _():
                    @pl.when(i + 1 >= 2)
                    def _():
                        pltpu.make_async_copy(idx_hbm.at[cid0], idx_v[bi1],
                                              sem_i[bi1]).wait()
                    pltpu.async_copy(x_hbm.at[idx_v[bi1].at[0, pl.ds(0, HCH)]],
                                     m_v[bn].at[pl.ds(0, HCH)], sem_g[bn])
                    pltpu.async_copy(x_hbm.at[idx_v[bi1].at[0, pl.ds(HCH, HCH)]],
                                     m_v[bn].at[pl.ds(HCH, HCH)], sem_g2[bn])
                    pltpu.async_copy(
                        e_hbm.at[pl.ds((cid0 + i + 1) * _CH, _CH)],
                        e_v[bn], sem_e[bn])

                    @pl.when(i + 2 < nit)
                    def _():
                        pltpu.async_copy(idx_hbm.at[cid0 + i + 2],
                                         idx_v[bi2], sem_i[bi2])

                @plsc.parallel_loop(0, _CH, 1, unroll=2)
                def _(j):
                    for k in range(ew // 16):
                        sl = pl.ds(k * 16, 16)
                        m_v[bm][j, sl] = jnp.maximum(
                            m_v[bm][j, sl] + e_v[bm][j, sl], 0.0)

                pltpu.async_copy(m_v[bm], acc_sh.at[idx_v[bi].at[1]],
                                 sem_s[bm], add=True)
            return carry

        lax.fori_loop(0, nit // 4, group, 0)
        pltpu.make_async_copy(m_v[(_NITA - 1) % 2],
                              acc_sh.at[idx_v[(_NITA - 1) % 4].at[1]],
                              sem_s[(_NITA - 1) % 2]).wait()
        plsc.subcore_barrier()
        pltpu.sync_copy(acc_sh.at[pl.ds(row0, _RPT)],
                        out_hbm.at[c, pl.ds(row0, _RPT)])

    return msgpass


def _node_mlp(x, aggr2, w1, b1, w2, b2, oc):
    BLK = 2000
    grid = _N // BLK

    def body(x_ref, a0_ref, a1_ref, w1_ref, b1_ref, w2_ref, b2_ref, o_ref):
        hsum = x_ref[...] + a0_ref[0] + a1_ref[0]
        t = jnp.maximum(
            jnp.dot(hsum, w1_ref[...], preferred_element_type=jnp.float32) + b1_ref[...], 0.0)
        o_ref[...] = jnp.maximum(
            jnp.dot(t, w2_ref[...], preferred_element_type=jnp.float32) + b2_ref[...], 0.0)

    def full(shape):
        return pl.BlockSpec(shape, lambda i: tuple(0 for _ in shape))

    return pl.pallas_call(
        body,
        grid=(grid,),
        in_specs=[pl.BlockSpec((BLK, _DF), lambda i: (i, 0)),
                  pl.BlockSpec((1, BLK, _DF), lambda i: (0, i, 0)),
                  pl.BlockSpec((1, BLK, _DF), lambda i: (1, i, 0)),
                  full((_DF, _H)), full((1, _H)),
                  full((_H, oc)), full((1, oc))],
        out_specs=pl.BlockSpec((BLK, oc), lambda i: (i, 0)),
        out_shape=jax.ShapeDtypeStruct((_N, oc), jnp.float32),
    )(x, aggr2, aggr2, w1, b1, w2, b2)


def _pool_classify(h, batch2d, cw1a, cw1b, cb1, cw2, cb2):
    BLK = 1000
    grid = _N // BLK
    NEG = -3.0e38

    def body(h_ref, b_ref, cw1a_ref, cw1b_ref, cb1_ref, cw2_ref, cb2_ref,
             out_ref, sum_ref, cnt_ref, mx_ref):
        i = pl.program_id(0)

        @pl.when(i == 0)
        def _():
            sum_ref[...] = jnp.zeros_like(sum_ref)
            cnt_ref[...] = jnp.zeros_like(cnt_ref)
            mx_ref[...] = jnp.full_like(mx_ref, NEG)

        hb = h_ref[...]
        bv = b_ref[...]
        onehot = (bv == lax.broadcasted_iota(jnp.int32, (BLK, _G), 1)
                  ).astype(jnp.float32)
        sum_ref[...] += lax.dot_general(
            onehot, hb, (((0,), (0,)), ((), ())), preferred_element_type=jnp.float32)
        cnt_ref[...] += lax.dot_general(
            onehot, jnp.ones_like(hb), (((0,), (0,)), ((), ())),
            preferred_element_type=jnp.float32)

        g0 = jnp.min(bv)
        g1 = jnp.max(bv)

        def mbody(g, carry):
            vals = jnp.where(bv == g, hb, NEG)
            m = jnp.max(vals, axis=0, keepdims=True)
            mx_ref[pl.ds(g, 1), :] = jnp.maximum(mx_ref[pl.ds(g, 1), :], m)
            return carry

        lax.fori_loop(g0, g1 + 1, mbody, 0)

        @pl.when(i == grid - 1)
        def _():
            cnt = cnt_ref[...]
            mean = sum_ref[...] / jnp.maximum(cnt, 1.0)
            mx = jnp.where(cnt > 0, mx_ref[...], 0.0)
            z = jnp.maximum(
                jnp.dot(mean, cw1a_ref[...], preferred_element_type=jnp.float32)
                + jnp.dot(mx, cw1b_ref[...], preferred_element_type=jnp.float32)
                + cb1_ref[...], 0.0)
            out_ref[...] = (jnp.dot(z, cw2_ref[...], preferred_element_type=jnp.float32)
                            + cb2_ref[...])

    def full(shape):
        return pl.BlockSpec(shape, lambda i: tuple(0 for _ in shape))

    return pl.pallas_call(
        body,
        grid=(grid,),
        in_specs=[pl.BlockSpec((BLK, _H), lambda i: (i, 0)),
                  pl.BlockSpec((BLK, 1), lambda i: (i, 0)),
                  full((_H, _H)), full((_H, _H)), full((1, _H)),
                  full((_H, 2)), full((1, 2))],
        out_specs=pl.BlockSpec((_G, 2), lambda i: (0, 0)),
        out_shape=jax.ShapeDtypeStruct((_G, 2), jnp.float32),
        scratch_shapes=[pltpu.VMEM((_G, _H), jnp.float32),
                        pltpu.VMEM((_G, _H), jnp.float32),
                        pltpu.VMEM((_G, _H), jnp.float32)],
    )(h, batch2d, cw1a, cw1b, cb1, cw2, cb2)


def kernel(x, edge_index, edge_attr, batch,
           c1_le_w, c1_le_b, c1_w1, c1_b1, c1_bn_g, c1_bn_b, c1_w2, c1_b2,
           c2_le_w, c2_le_b, c2_w1, c2_b1, c2_bn_g, c2_bn_b, c2_w2, c2_b2,
           c3_le_w, c3_le_b, c3_w1, c3_b1, c3_bn_g, c3_bn_b, c3_w2, c3_b2,
           cl_w1, cl_b1, cl_w2, cl_b2):
    padE = _EPAD - _E
    src_p = jnp.concatenate([edge_index[0], jnp.zeros((padE,), jnp.int32)])
    dst_p = jnp.concatenate([edge_index[1], jnp.full((padE,), _N, jnp.int32)])
    ea_p = jnp.concatenate([edge_attr, jnp.zeros((padE, _DE), jnp.float32)])
    idx2 = jnp.stack([src_p.reshape(-1, _CH), dst_p.reshape(-1, _CH)], axis=1)

    bnscale = jnp.float32(1.0 / (1.0 + _BN_EPS) ** 0.5)

    def padcols(w, n):
        return jnp.pad(w, ((0, 0), (0, n - w.shape[-1])))

    def padrows(w, n):
        return jnp.pad(w, ((0, n - w.shape[0]), (0, 0)))

    layers = []
    for li, (le_w, le_b, w1, b1, bn_g, bn_b, w2, b2) in enumerate((
            (c1_le_w, c1_le_b, c1_w1, c1_b1, c1_bn_g, c1_bn_b, c1_w2, c1_b2),
            (c2_le_w, c2_le_b, c2_w1, c2_b1, c2_bn_g, c2_bn_b, c2_w2, c2_b2),
            (c3_le_w, c3_le_b, c3_w1, c3_b1, c3_bn_g, c3_bn_b, c3_w2, c3_b2))):
        g = bn_g * bnscale
        w1f = w1 * g[None, :]
        b1f = (b1 * g + bn_b).reshape(1, -1)
        oc = _DF if li < 2 else _H
        ew = _DF if li == 0 else _H
        layers.append((padcols(le_w, ew), padcols(le_b.reshape(1, -1), ew),
                       padrows(w1f, _DF), b1f,
                       padcols(w2, oc), padcols(b2.reshape(1, -1), oc), oc, ew))

    e1, e2, e3 = _edge_linear(ea_p, [(layers[0][0], layers[0][1]),
                                     (layers[1][0], layers[1][1]),
                                     (layers[2][0], layers[2][1])])

    zeros = jnp.zeros((_NP, _DF), jnp.float32)
    h = x
    for (e_l, lay) in ((e1, layers[0]), (e2, layers[1]), (e3, layers[2])):
        aggr2 = _make_msgpass(lay[7])(idx2, e_l, h, zeros)
        h = _node_mlp(h, aggr2, lay[2], lay[3], lay[4], lay[5], lay[6])

    return _pool_classify(h, batch.reshape(_N, 1),
                          cl_w1[:_H], cl_w1[_H:], cl_b1.reshape(1, -1),
                          cl_w2, cl_b2.reshape(1, -1))


# speedup vs baseline: 1.1573x; 1.0194x over previous
import functools

import jax
import jax.numpy as jnp
from jax import lax
from jax.experimental import pallas as pl
from jax.experimental.pallas import tpu as pltpu
from jax.experimental.pallas import tpu_sc as plsc

_N = 10000
_E = 320000
_DF = 128
_DE = 16
_H = 96
_G = 64
_BN_EPS = 1e-5

_TILES = 32
_CH = 80
_EPT = 10240
_EPAD = _TILES * _EPT
_NIT = _EPT // _CH
_NITA = 192
_NITB = 2 * _NIT - _NITA
_NP = 10112
_RPT = _NP // 16


def _edge_linear(ea, wbs):
    BLK = 4096
    grid = _EPAD // BLK

    def body(ea_ref, w_ref, b_ref, o_ref):
        o_ref[...] = jnp.dot(ea_ref[...], w_ref[...],
                             preferred_element_type=jnp.float32) + b_ref[...]

    def full(shape):
        return pl.BlockSpec(shape, lambda i: (0, 0))

    outs = []
    for (w, b) in wbs:
        ew = w.shape[1]
        outs.append(pl.pallas_call(
            body,
            grid=(grid,),
            in_specs=[pl.BlockSpec((BLK, _DE), lambda i: (i, 0)),
                      full((_DE, ew)), full((1, ew))],
            out_specs=pl.BlockSpec((BLK, ew), lambda i: (i, 0)),
            out_shape=jax.ShapeDtypeStruct((_EPAD, ew), jnp.float32),
        )(ea, w, b))
    return outs


def _make_msgpass(ew):
    mesh = plsc.VectorSubcoreMesh(core_axis_name="c", subcore_axis_name="s")
    HCH = _CH // 2

    @functools.partial(
        pl.kernel,
        out_type=jax.ShapeDtypeStruct((2, _NP, _DF), jnp.float32),
        mesh=mesh,
        compiler_params=pltpu.CompilerParams(use_tc_tiling_on_sc=True),
        scratch_types=[
            [pltpu.VMEM((2, _CH), jnp.int32) for _ in range(4)],
            [pltpu.VMEM((_CH, _DF), jnp.float32) for _ in range(2)],
            [pltpu.VMEM((_CH, ew), jnp.float32) for _ in range(2)],
            pltpu.VMEM_SHARED((_NP, _DF), jnp.float32),
            [pltpu.SemaphoreType.DMA for _ in range(4)],
            [pltpu.SemaphoreType.DMA for _ in range(2)],
            [pltpu.SemaphoreType.DMA for _ in range(2)],
            [pltpu.SemaphoreType.DMA for _ in range(2)],
            [pltpu.SemaphoreType.DMA for _ in range(2)],
        ],
    )
    def msgpass(idx_hbm, e_hbm, x_hbm, z_hbm, out_hbm,
                idx_v, m_v, e_v, acc_sh, sem_i, sem_e, sem_g, sem_g2, sem_s):
        c = lax.axis_index("c")
        s = lax.axis_index("s")
        row0 = s * _RPT
        nit = jnp.where(c == 0, _NITA, _NITB)
        cid0 = jnp.where(c == 0, s * _NITA, 16 * _NITA + s * _NITB)
        pltpu.sync_copy(z_hbm.at[pl.ds(row0, _RPT)], acc_sh.at[pl.ds(row0, _RPT)])
        plsc.subcore_barrier()

        pltpu.sync_copy(idx_hbm.at[cid0], idx_v[0])
        pltpu.sync_copy(idx_hbm.at[cid0 + 1], idx_v[1])
        pltpu.async_copy(e_hbm.at[pl.ds(cid0 * _CH, _CH)], e_v[0], sem_e[0])
        pltpu.async_copy(x_hbm.at[idx_v[0].at[0, pl.ds(0, HCH)]],
                         m_v[0].at[pl.ds(0, HCH)], sem_g[0])
        pltpu.async_copy(x_hbm.at[idx_v[0].at[0, pl.ds(HCH, HCH)]],
                         m_v[0].at[pl.ds(HCH, HCH)], sem_g2[0])

        def group(g, carry):
            for u in range(4):
                i = g * 4 + u
                bm = u % 2
                bn = (u + 1) % 2
                bi = u % 4
                bi1 = (u + 1) % 4
                bi2 = (u + 2) % 4
                bi_1 = (u + 3) % 4
                pltpu.make_async_copy(e_hbm.at[pl.ds(0, _CH)], e_v[bm],
                                      sem_e[bm]).wait()
                pltpu.make_async_copy(x_hbm.at[idx_v[bi].at[0, pl.ds(0, HCH)]],
                                      m_v[bm].at[pl.ds(0, HCH)],
                                      sem_g[bm]).wait()
                pltpu.make_async_copy(x_hbm.at[idx_v[bi].at[0, pl.ds(0, HCH)]],
                                      m_v[bm].at[pl.ds(HCH, HCH)],
                                      sem_g2[bm]).wait()

                @pl.when(i >= 1)
                def _():
                    pltpu.make_async_copy(m_v[bn], acc_sh.at[idx_v[bi_1].at[1]],
                                          sem_s[bn]).wait()

                @pl.when(i + 1 < nit)
                def _():
                    @pl.when(i + 1 >= 2)
                    def _():
                        pltpu.make_async_copy(idx_hbm.at[cid0], idx_v[bi1],
                                              sem_i[bi1]).wait()
                    pltpu.async_copy(x_hbm.at[idx_v[bi1].at[0, pl.ds(0, HCH)]],
                                     m_v[bn].at[pl.ds(0, HCH)], sem_g[bn])
                    pltpu.async_copy(x_hbm.at[idx_v[bi1].at[0, pl.ds(HCH, HCH)]],
                                     m_v[bn].at[pl.ds(HCH, HCH)], sem_g2[bn])
                    pltpu.async_copy(
                        e_hbm.at[pl.ds((cid0 + i + 1) * _CH, _CH)],
                        e_v[bn], sem_e[bn])

                    @pl.when(i + 2 < nit)
                    def _():
                        pltpu.async_copy(idx_hbm.at[cid0 + i + 2],
                                         idx_v[bi2], sem_i[bi2])

                @plsc.parallel_loop(0, _CH, 1, unroll=2)
                def _(j):
                    for k in range(ew // 16):
                        sl = pl.ds(k * 16, 16)
                        m_v[bm][j, sl] = jnp.maximum(
                            m_v[bm][j, sl] + e_v[bm][j, sl], 0.0)

                pltpu.async_copy(m_v[bm], acc_sh.at[idx_v[bi].at[1]],
                                 sem_s[bm], add=True)
            return carry

        lax.fori_loop(0, nit // 4, group, 0)
        pltpu.make_async_copy(m_v[(_NITA - 1) % 2],
                              acc_sh.at[idx_v[(_NITA - 1) % 4].at[1]],
                              sem_s[(_NITA - 1) % 2]).wait()
        plsc.subcore_barrier()
        pltpu.sync_copy(acc_sh.at[pl.ds(row0, _RPT)],
                        out_hbm.at[c, pl.ds(row0, _RPT)])

    return msgpass


def _node_mlp(x, aggr2, w1, b1, w2, b2, oc):
    BLK = 2000
    grid = _N // BLK

    def body(x_ref, a0_ref, a1_ref, w1_ref, b1_ref, w2_ref, b2_ref, o_ref):
        hsum = x_ref[...] + a0_ref[0] + a1_ref[0]
        t = jnp.maximum(
            jnp.dot(hsum, w1_ref[...], preferred_element_type=jnp.float32) + b1_ref[...], 0.0)
        o_ref[...] = jnp.maximum(
            jnp.dot(t, w2_ref[...], preferred_element_type=jnp.float32) + b2_ref[...], 0.0)

    def full(shape):
        return pl.BlockSpec(shape, lambda i: tuple(0 for _ in shape))

    return pl.pallas_call(
        body,
        grid=(grid,),
        in_specs=[pl.BlockSpec((BLK, _DF), lambda i: (i, 0)),
                  pl.BlockSpec((1, BLK, _DF), lambda i: (0, i, 0)),
                  pl.BlockSpec((1, BLK, _DF), lambda i: (1, i, 0)),
                  full((_DF, _H)), full((1, _H)),
                  full((_H, oc)), full((1, oc))],
        out_specs=pl.BlockSpec((BLK, oc), lambda i: (i, 0)),
        out_shape=jax.ShapeDtypeStruct((_N, oc), jnp.float32),
    )(x, aggr2, aggr2, w1, b1, w2, b2)


def _pool_classify(h, batch2d, cw1a, cw1b, cb1, cw2, cb2):
    BLK = 1000
    grid = _N // BLK
    NEG = -3.0e38

    def body(h_ref, b_ref, cw1a_ref, cw1b_ref, cb1_ref, cw2_ref, cb2_ref,
             out_ref, sum_ref, cnt_ref, mx_ref):
        i = pl.program_id(0)

        @pl.when(i == 0)
        def _():
            sum_ref[...] = jnp.zeros_like(sum_ref)
            cnt_ref[...] = jnp.zeros_like(cnt_ref)
            mx_ref[...] = jnp.full_like(mx_ref, NEG)

        hb = h_ref[...]
        bv = b_ref[...]
        onehot = (bv == lax.broadcasted_iota(jnp.int32, (BLK, _G), 1)
                  ).astype(jnp.float32)
        sum_ref[...] += lax.dot_general(
            onehot, hb, (((0,), (0,)), ((), ())), preferred_element_type=jnp.float32)
        cnt_ref[...] += lax.dot_general(
            onehot, jnp.ones_like(hb), (((0,), (0,)), ((), ())),
            preferred_element_type=jnp.float32)

        g0 = jnp.min(bv)
        g1 = jnp.max(bv)

        def mbody(g, carry):
            vals = jnp.where(bv == g, hb, NEG)
            m = jnp.max(vals, axis=0, keepdims=True)
            mx_ref[pl.ds(g, 1), :] = jnp.maximum(mx_ref[pl.ds(g, 1), :], m)
            return carry

        lax.fori_loop(g0, g1 + 1, mbody, 0)

        @pl.when(i == grid - 1)
        def _():
            cnt = cnt_ref[...]
            mean = sum_ref[...] / jnp.maximum(cnt, 1.0)
            mx = jnp.where(cnt > 0, mx_ref[...], 0.0)
            z = jnp.maximum(
                jnp.dot(mean, cw1a_ref[...], preferred_element_type=jnp.float32)
                + jnp.dot(mx, cw1b_ref[...], preferred_element_type=jnp.float32)
                + cb1_ref[...], 0.0)
            out_ref[...] = (jnp.dot(z, cw2_ref[...], preferred_element_type=jnp.float32)
                            + cb2_ref[...])

    def full(shape):
        return pl.BlockSpec(shape, lambda i: tuple(0 for _ in shape))

    return pl.pallas_call(
        body,
        grid=(grid,),
        in_specs=[pl.BlockSpec((BLK, _H), lambda i: (i, 0)),
                  pl.BlockSpec((BLK, 1), lambda i: (i, 0)),
                  full((_H, _H)), full((_H, _H)), full((1, _H)),
                  full((_H, 2)), full((1, 2))],
        out_specs=pl.BlockSpec((_G, 2), lambda i: (0, 0)),
        out_shape=jax.ShapeDtypeStruct((_G, 2), jnp.float32),
        scratch_shapes=[pltpu.VMEM((_G, _H), jnp.float32),
                        pltpu.VMEM((_G, _H), jnp.float32),
                        pltpu.VMEM((_G, _H), jnp.float32)],
    )(h, batch2d, cw1a, cw1b, cb1, cw2, cb2)


def kernel(x, edge_index, edge_attr, batch,
           c1_le_w, c1_le_b, c1_w1, c1_b1, c1_bn_g, c1_bn_b, c1_w2, c1_b2,
           c2_le_w, c2_le_b, c2_w1, c2_b1, c2_bn_g, c2_bn_b, c2_w2, c2_b2,
           c3_le_w, c3_le_b, c3_w1, c3_b1, c3_bn_g, c3_bn_b, c3_w2, c3_b2,
           cl_w1, cl_b1, cl_w2, cl_b2):
    padE = _EPAD - _E
    src_p = jnp.concatenate([edge_index[0], jnp.zeros((padE,), jnp.int32)])
    dst_p = jnp.concatenate([edge_index[1], jnp.full((padE,), _N, jnp.int32)])
    ea_p = jnp.concatenate([edge_attr, jnp.zeros((padE, _DE), jnp.float32)])
    idx2 = jnp.stack([src_p.reshape(-1, _CH), dst_p.reshape(-1, _CH)], axis=1)

    bnscale = jnp.float32(1.0 / (1.0 + _BN_EPS) ** 0.5)

    def padcols(w, n):
        return jnp.pad(w, ((0, 0), (0, n - w.shape[-1])))

    def padrows(w, n):
        return jnp.pad(w, ((0, n - w.shape[0]), (0, 0)))

    layers = []
    for li, (le_w, le_b, w1, b1, bn_g, bn_b, w2, b2) in enumerate((
            (c1_le_w, c1_le_b, c1_w1, c1_b1, c1_bn_g, c1_bn_b, c1_w2, c1_b2),
            (c2_le_w, c2_le_b, c2_w1, c2_b1, c2_bn_g, c2_bn_b, c2_w2, c2_b2),
            (c3_le_w, c3_le_b, c3_w1, c3_b1, c3_bn_g, c3_bn_b, c3_w2, c3_b2))):
        g = bn_g * bnscale
        w1f = w1 * g[None, :]
        b1f = (b1 * g + bn_b).reshape(1, -1)
        oc = _DF if li < 2 else _H
        ew = _DF if li == 0 else _H
        layers.append((padcols(le_w, ew), padcols(le_b.reshape(1, -1), ew),
                       padrows(w1f, _DF), b1f,
                       padcols(w2, oc), padcols(b2.reshape(1, -1), oc), oc, ew))

    e1, e2, e3 = _edge_linear(ea_p, [(layers[0][0], layers[0][1]),
                                     (layers[1][0], layers[1][1]),
                                     (layers[2][0], layers[2][1])])

    zeros = jnp.zeros((_NP, _DF), jnp.float32)
    h = x
    for (e_l, lay) in ((e1, layers[0]), (e2, layers[1]), (e3, layers[2])):
        aggr2 = _make_msgpass(lay[7])(idx2, e_l, h, zeros)
        h = _node_mlp(h, aggr2, lay[2], lay[3], lay[4], lay[5], lay[6])

    return _pool_classify(h, batch.reshape(_N, 1),
                          cl_w1[:_H], cl_w1[_H:], cl_b1.reshape(1, -1),
                          cl_w2, cl_b2.reshape(1, -1))


# speedup vs baseline: 1.2954x; 1.1193x over previous
import functools

import jax
import jax.numpy as jnp
from jax import lax
from jax.experimental import pallas as pl
from jax.experimental.pallas import tpu as pltpu
from jax.experimental.pallas import tpu_sc as plsc

_N = 10000
_E = 320000
_DF = 128
_DE = 16
_H = 96
_G = 64
_BN_EPS = 1e-5

_TILES = 32
_CH = 80
_EPT = 10240
_EPAD = _TILES * _EPT
_NIT = _EPT // _CH
_NITA = 208
_NITB = 2 * _NIT - _NITA
_NP = 10112
_RPT = _NP // 16


def _edge_linear(ea, wbs):
    BLK = 4096
    grid = _EPAD // BLK

    def body(ea_ref, w_ref, b_ref, o_ref):
        o_ref[...] = jnp.dot(ea_ref[...], w_ref[...],
                             preferred_element_type=jnp.float32) + b_ref[...]

    def full(shape):
        return pl.BlockSpec(shape, lambda i: (0, 0))

    outs = []
    for (w, b) in wbs:
        ew = w.shape[1]
        outs.append(pl.pallas_call(
            body,
            grid=(grid,),
            in_specs=[pl.BlockSpec((BLK, _DE), lambda i: (i, 0)),
                      full((_DE, ew)), full((1, ew))],
            out_specs=pl.BlockSpec((BLK, ew), lambda i: (i, 0)),
            out_shape=jax.ShapeDtypeStruct((_EPAD, ew), jnp.float32),
        )(ea, w, b))
    return outs


def _make_msgpass(ew):
    mesh = plsc.VectorSubcoreMesh(core_axis_name="c", subcore_axis_name="s")
    HCH = _CH // 2

    @functools.partial(
        pl.kernel,
        out_type=jax.ShapeDtypeStruct((2, _NP, _DF), jnp.float32),
        mesh=mesh,
        compiler_params=pltpu.CompilerParams(use_tc_tiling_on_sc=True),
        scratch_types=[
            [pltpu.VMEM((2, _CH), jnp.int32) for _ in range(4)],
            [pltpu.VMEM((_CH, _DF), jnp.float32) for _ in range(2)],
            [pltpu.VMEM((_CH, ew), jnp.float32) for _ in range(2)],
            pltpu.VMEM_SHARED((_NP, _DF), jnp.float32),
            [pltpu.SemaphoreType.DMA for _ in range(4)],
            [pltpu.SemaphoreType.DMA for _ in range(2)],
            [pltpu.SemaphoreType.DMA for _ in range(2)],
            [pltpu.SemaphoreType.DMA for _ in range(2)],
            [pltpu.SemaphoreType.DMA for _ in range(2)],
        ],
    )
    def msgpass(idx_hbm, e_hbm, x_hbm, z_hbm, out_hbm,
                idx_v, m_v, e_v, acc_sh, sem_i, sem_e, sem_g, sem_g2, sem_s):
        c = lax.axis_index("c")
        s = lax.axis_index("s")
        row0 = s * _RPT
        nit = jnp.where(c == 0, _NITA, _NITB)
        cid0 = jnp.where(c == 0, s * _NITA, 16 * _NITA + s * _NITB)
        pltpu.sync_copy(z_hbm.at[pl.ds(row0, _RPT)], acc_sh.at[pl.ds(row0, _RPT)])
        plsc.subcore_barrier()

        pltpu.sync_copy(idx_hbm.at[cid0], idx_v[0])
        pltpu.sync_copy(idx_hbm.at[cid0 + 1], idx_v[1])
        pltpu.async_copy(e_hbm.at[pl.ds(cid0 * _CH, _CH)], e_v[0], sem_e[0])
        pltpu.async_copy(x_hbm.at[idx_v[0].at[0, pl.ds(0, HCH)]],
                         m_v[0].at[pl.ds(0, HCH)], sem_g[0])
        pltpu.async_copy(x_hbm.at[idx_v[0].at[0, pl.ds(HCH, HCH)]],
                         m_v[0].at[pl.ds(HCH, HCH)], sem_g2[0])

        def group(g, carry):
            for u in range(4):
                i = g * 4 + u
                bm = u % 2
                bn = (u + 1) % 2
                bi = u % 4
                bi1 = (u + 1) % 4
                bi2 = (u + 2) % 4
                bi_1 = (u + 3) % 4
                pltpu.make_async_copy(e_hbm.at[pl.ds(0, _CH)], e_v[bm],
                                      sem_e[bm]).wait()
                pltpu.make_async_copy(x_hbm.at[idx_v[bi].at[0, pl.ds(0, HCH)]],
                                      m_v[bm].at[pl.ds(0, HCH)],
                                      sem_g[bm]).wait()
                pltpu.make_async_copy(x_hbm.at[idx_v[bi].at[0, pl.ds(0, HCH)]],
                                      m_v[bm].at[pl.ds(HCH, HCH)],
                                      sem_g2[bm]).wait()

                @pl.when(i >= 1)
                def _():
                    pltpu.make_async_copy(m_v[bn], acc_sh.at[idx_v[bi_1].at[1]],
                                          sem_s[bn]).wait()

                @pl.when(i + 1 < nit)
                def _():
                    @pl.when(i + 1 >= 2)
                    def _():
                        pltpu.make_async_copy(idx_hbm.at[cid0], idx_v[bi1],
                                              sem_i[bi1]).wait()
                    pltpu.async_copy(x_hbm.at[idx_v[bi1].at[0, pl.ds(0, HCH)]],
                                     m_v[bn].at[pl.ds(0, HCH)], sem_g[bn])
                    pltpu.async_copy(x_hbm.at[idx_v[bi1].at[0, pl.ds(HCH, HCH)]],
                                     m_v[bn].at[pl.ds(HCH, HCH)], sem_g2[bn])
                    pltpu.async_copy(
                        e_hbm.at[pl.ds((cid0 + i + 1) * _CH, _CH)],
                        e_v[bn], sem_e[bn])

                    @pl.when(i + 2 < nit)
                    def _():
                        pltpu.async_copy(idx_hbm.at[cid0 + i + 2],
                                         idx_v[bi2], sem_i[bi2])

                @plsc.parallel_loop(0, _CH, 1, unroll=2)
                def _(j):
                    for k in range(ew // 16):
                        sl = pl.ds(k * 16, 16)
                        m_v[bm][j, sl] = jnp.maximum(
                            m_v[bm][j, sl] + e_v[bm][j, sl], 0.0)

                pltpu.async_copy(m_v[bm], acc_sh.at[idx_v[bi].at[1]],
                                 sem_s[bm], add=True)
            return carry

        lax.fori_loop(0, nit // 4, group, 0)
        pltpu.make_async_copy(m_v[(_NITA - 1) % 2],
                              acc_sh.at[idx_v[(_NITA - 1) % 4].at[1]],
                              sem_s[(_NITA - 1) % 2]).wait()
        plsc.subcore_barrier()
        pltpu.sync_copy(acc_sh.at[pl.ds(row0, _RPT)],
                        out_hbm.at[c, pl.ds(row0, _RPT)])

    return msgpass


def _node_mlp(x, aggr2, w1, b1, w2, b2, oc):
    BLK = 2000
    grid = _N // BLK

    def body(x_ref, a0_ref, a1_ref, w1_ref, b1_ref, w2_ref, b2_ref, o_ref):
        hsum = x_ref[...] + a0_ref[0] + a1_ref[0]
        t = jnp.maximum(
            jnp.dot(hsum, w1_ref[...], preferred_element_type=jnp.float32) + b1_ref[...], 0.0)
        o_ref[...] = jnp.maximum(
            jnp.dot(t, w2_ref[...], preferred_element_type=jnp.float32) + b2_ref[...], 0.0)

    def full(shape):
        return pl.BlockSpec(shape, lambda i: tuple(0 for _ in shape))

    return pl.pallas_call(
        body,
        grid=(grid,),
        in_specs=[pl.BlockSpec((BLK, _DF), lambda i: (i, 0)),
                  pl.BlockSpec((1, BLK, _DF), lambda i: (0, i, 0)),
                  pl.BlockSpec((1, BLK, _DF), lambda i: (1, i, 0)),
                  full((_DF, _H)), full((1, _H)),
                  full((_H, oc)), full((1, oc))],
        out_specs=pl.BlockSpec((BLK, oc), lambda i: (i, 0)),
        out_shape=jax.ShapeDtypeStruct((_N, oc), jnp.float32),
    )(x, aggr2, aggr2, w1, b1, w2, b2)


def _pool_classify(h, batch2d, cw1a, cw1b, cb1, cw2, cb2):
    BLK = 1000
    grid = _N // BLK
    NEG = -3.0e38

    def body(h_ref, b_ref, cw1a_ref, cw1b_ref, cb1_ref, cw2_ref, cb2_ref,
             out_ref, sum_ref, cnt_ref, mx_ref):
        i = pl.program_id(0)

        @pl.when(i == 0)
        def _():
            sum_ref[...] = jnp.zeros_like(sum_ref)
            cnt_ref[...] = jnp.zeros_like(cnt_ref)
            mx_ref[...] = jnp.full_like(mx_ref, NEG)

        hb = h_ref[...]
        bv = b_ref[...]
        onehot = (bv == lax.broadcasted_iota(jnp.int32, (BLK, _G), 1)
                  ).astype(jnp.float32)
        sum_ref[...] += lax.dot_general(
            onehot, hb, (((0,), (0,)), ((), ())), preferred_element_type=jnp.float32)
        cnt_ref[...] += lax.dot_general(
            onehot, jnp.ones_like(hb), (((0,), (0,)), ((), ())),
            preferred_element_type=jnp.float32)

        g0 = jnp.min(bv)
        g1 = jnp.max(bv)

        def mbody(g, carry):
            vals = jnp.where(bv == g, hb, NEG)
            m = jnp.max(vals, axis=0, keepdims=True)
            mx_ref[pl.ds(g, 1), :] = jnp.maximum(mx_ref[pl.ds(g, 1), :], m)
            return carry

        lax.fori_loop(g0, g1 + 1, mbody, 0)

        @pl.when(i == grid - 1)
        def _():
            cnt = cnt_ref[...]
            mean = sum_ref[...] / jnp.maximum(cnt, 1.0)
            mx = jnp.where(cnt > 0, mx_ref[...], 0.0)
            z = jnp.maximum(
                jnp.dot(mean, cw1a_ref[...], preferred_element_type=jnp.float32)
                + jnp.dot(mx, cw1b_ref[...], preferred_element_type=jnp.float32)
                + cb1_ref[...], 0.0)
            out_ref[...] = (jnp.dot(z, cw2_ref[...], preferred_element_type=jnp.float32)
                            + cb2_ref[...])

    def full(shape):
        return pl.BlockSpec(shape, lambda i: tuple(0 for _ in shape))

    return pl.pallas_call(
        body,
        grid=(grid,),
        in_specs=[pl.BlockSpec((BLK, _H), lambda i: (i, 0)),
                  pl.BlockSpec((BLK, 1), lambda i: (i, 0)),
                  full((_H, _H)), full((_H, _H)), full((1, _H)),
                  full((_H, 2)), full((1, 2))],
        out_specs=pl.BlockSpec((_G, 2), lambda i: (0, 0)),
        out_shape=jax.ShapeDtypeStruct((_G, 2), jnp.float32),
        scratch_shapes=[pltpu.VMEM((_G, _H), jnp.float32),
                        pltpu.VMEM((_G, _H), jnp.float32),
                        pltpu.VMEM((_G, _H), jnp.float32)],
    )(h, batch2d, cw1a, cw1b, cb1, cw2, cb2)


def kernel(x, edge_index, edge_attr, batch,
           c1_le_w, c1_le_b, c1_w1, c1_b1, c1_bn_g, c1_bn_b, c1_w2, c1_b2,
           c2_le_w, c2_le_b, c2_w1, c2_b1, c2_bn_g, c2_bn_b, c2_w2, c2_b2,
           c3_le_w, c3_le_b, c3_w1, c3_b1, c3_bn_g, c3_bn_b, c3_w2, c3_b2,
           cl_w1, cl_b1, cl_w2, cl_b2):
    padE = _EPAD - _E
    src_p = jnp.concatenate([edge_index[0], jnp.zeros((padE,), jnp.int32)])
    dst_p = jnp.concatenate([edge_index[1], jnp.full((padE,), _N, jnp.int32)])
    ea_p = jnp.concatenate([edge_attr, jnp.zeros((padE, _DE), jnp.float32)])
    idx2 = jnp.stack([src_p.reshape(-1, _CH), dst_p.reshape(-1, _CH)], axis=1)

    bnscale = jnp.float32(1.0 / (1.0 + _BN_EPS) ** 0.5)

    def padcols(w, n):
        return jnp.pad(w, ((0, 0), (0, n - w.shape[-1])))

    def padrows(w, n):
        return jnp.pad(w, ((0, n - w.shape[0]), (0, 0)))

    layers = []
    for li, (le_w, le_b, w1, b1, bn_g, bn_b, w2, b2) in enumerate((
            (c1_le_w, c1_le_b, c1_w1, c1_b1, c1_bn_g, c1_bn_b, c1_w2, c1_b2),
            (c2_le_w, c2_le_b, c2_w1, c2_b1, c2_bn_g, c2_bn_b, c2_w2, c2_b2),
            (c3_le_w, c3_le_b, c3_w1, c3_b1, c3_bn_g, c3_bn_b, c3_w2, c3_b2))):
        g = bn_g * bnscale
        w1f = w1 * g[None, :]
        b1f = (b1 * g + bn_b).reshape(1, -1)
        oc = _DF if li < 2 else _H
        ew = _DF if li == 0 else _H
        layers.append((padcols(le_w, ew), padcols(le_b.reshape(1, -1), ew),
                       padrows(w1f, _DF), b1f,
                       padcols(w2, oc), padcols(b2.reshape(1, -1), oc), oc, ew))

    e1, e2, e3 = _edge_linear(ea_p, [(layers[0][0], layers[0][1]),
                                     (layers[1][0], layers[1][1]),
                                     (layers[2][0], layers[2][1])])

    zeros = jnp.zeros((_NP, _DF), jnp.float32)
    h = x
    for (e_l, lay) in ((e1, layers[0]), (e2, layers[1]), (e3, layers[2])):
        aggr2 = _make_msgpass(lay[7])(idx2, e_l, h, zeros)
        h = _node_mlp(h, aggr2, lay[2], lay[3], lay[4], lay[5], lay[6])

    return _pool_classify(h, batch.reshape(_N, 1),
                          cl_w1[:_H], cl_w1[_H:], cl_b1.reshape(1, -1),
                          cl_w2, cl_b2.reshape(1, -1))
